# baseline (speedup 1.0000x reference)
import functools

import jax
import jax.numpy as jnp
from jax import lax
from jax.experimental import pallas as pl
from jax.experimental.pallas import tpu as pltpu
from jax.experimental.pallas import tpu_sc as plsc

N = 10000
E = 160000
D = 256
H = 256
C = 40
CP = 64

NC = 2
NS = 16
DEGP = 10240
EB3, NB3 = 40, 5
EB5, NB5 = 80, 5

RB = 400
NRB = N // RB

_mesh = plsc.VectorSubcoreMesh(core_axis_name="c", subcore_axis_name="s")

_sc_params = pltpu.CompilerParams(
    needs_layout_passes=False, use_tc_tiling_on_sc=False)


def _deg_body(dst_hbm, zdeg_hbm, out_hbm, idx_v, deg_v):
    c = lax.axis_index("c")
    s = lax.axis_index("s")
    wid = c * NS + s
    n_edges = E // (NC * NS)
    pltpu.sync_copy(zdeg_hbm, deg_v)
    idx_v[pl.ds(n_edges - 8, 16)] = jnp.zeros((16,), jnp.int32)
    pltpu.sync_copy(dst_hbm.at[pl.ds(wid * n_edges, n_edges)],
                    idx_v.at[pl.ds(0, n_edges)])
    ones = jnp.ones((16,), jnp.float32)
    iota = lax.iota(jnp.int32, 16)
    nblk = (n_edges + 15) // 16

    @pl.loop(0, nblk)
    def _(k):
        idx = idx_v[pl.ds(k * 16, 16)]
        mask = (k * 16 + iota) < n_edges
        plsc.addupdate_scatter(deg_v, [idx], ones, mask=mask)

    pltpu.sync_copy(deg_v, out_hbm.at[wid])


def _deg_kernel(dst32, zdeg):
    k = pl.kernel(
        _deg_body,
        out_type=jax.ShapeDtypeStruct((NC * NS, DEGP), jnp.float32),
        mesh=_mesh,
        compiler_params=_sc_params,
        scratch_types=[
            pltpu.VMEM((E // (NC * NS) + 8, ), jnp.int32),
            pltpu.VMEM((DEGP,), jnp.float32),
        ],
    )
    return k(dst32, zdeg)


def _make_scatter(F, eb, nb):
    rows_per_tile = N // NS
    ept = E // NS
    nblk = ept // eb
    assert nblk * eb == ept and nblk % nb == 0 and eb % 8 == 0

    def body(y_hbm, src_hbm, dst_hbm, out_hbm,
             src_v, dst_v, rows_v, acc_sh, *sems):
        c = lax.axis_index("c")
        s = lax.axis_index("s")
        yc = y_hbm.at[c]
        pltpu.sync_copy(yc.at[pl.ds(s * rows_per_tile, rows_per_tile)],
                        acc_sh.at[pl.ds(s * rows_per_tile, rows_per_tile)])
        pltpu.sync_copy(src_hbm.at[pl.ds(s * ept, ept)], src_v)
        pltpu.sync_copy(dst_hbm.at[pl.ds(s * ept, ept)], dst_v)
        plsc.subcore_barrier()

        gsem, ssem = sems[:nb], sems[nb:]
        for b in range(nb):
            pltpu.async_copy(
                yc.at[src_v.at[pl.ds(b * eb, eb)]], rows_v.at[b], gsem[b])

        @pl.loop(0, nblk, step=nb)
        def _(j0):
            for b in range(nb):
                j = j0 + b
                pltpu.make_async_copy(
                    yc.at[src_v.at[pl.ds(j * eb, eb)]], rows_v.at[b],
                    gsem[b]).wait()
                pltpu.async_copy(rows_v.at[b],
                                 acc_sh.at[dst_v.at[pl.ds(j * eb, eb)]],
                                 ssem[b], add=True)
            for b in range(nb):
                j = j0 + b

                @pl.when(j + nb < nblk)
                def _():
                    pltpu.make_async_copy(
                        rows_v.at[b],
                        acc_sh.at[dst_v.at[pl.ds(j * eb, eb)]],
                        ssem[b]).wait()
                    pltpu.async_copy(
                        yc.at[src_v.at[pl.ds((j + nb) * eb, eb)]],
                        rows_v.at[b], gsem[b])

        for b in range(nb):
            pltpu.make_async_copy(
                rows_v.at[b],
                acc_sh.at[dst_v.at[pl.ds((nblk - nb + b) * eb, eb)]],
                ssem[b]).wait()
        plsc.subcore_barrier()
        pltpu.sync_copy(
            acc_sh.at[pl.ds(s * rows_per_tile, rows_per_tile)],
            out_hbm.at[c].at[pl.ds(s * rows_per_tile, rows_per_tile)])

    def run(y3, src, dst):
        k = pl.kernel(
            body,
            out_type=jax.ShapeDtypeStruct((NC, N, F), jnp.float32),
            mesh=_mesh,
            compiler_params=_sc_params,
            scratch_types=[
                pltpu.VMEM((ept,), jnp.int32),
                pltpu.VMEM((ept,), jnp.int32),
                pltpu.VMEM((nb, eb, F), jnp.float32),
                pltpu.VMEM_SHARED((N, F), jnp.float32),
            ] + [pltpu.SemaphoreType.DMA] * (2 * nb),
        )
        return k(y3, src, dst)

    return run


_scatter128 = _make_scatter(128, EB3, NB3)
_scatter32 = _make_scatter(32, EB5, NB5)


def _k2_body(x_ref, w1_ref, degp_ref, y_ref, dinv_ref):
    deg = jnp.sum(degp_ref[...], axis=1, keepdims=True) + 1.0
    dinv = lax.rsqrt(deg)
    xw = jnp.dot(x_ref[...], w1_ref[...], preferred_element_type=jnp.float32)
    y_ref[...] = (xw * dinv)[None]
    dinv_ref[...] = jnp.broadcast_to(dinv, (RB, 128))


def _k2(x, W1, degp):
    return pl.pallas_call(
        _k2_body,
        grid=(NRB, 2),
        in_specs=[
            pl.BlockSpec((RB, D), lambda i, c: (i, 0)),
            pl.BlockSpec((D, 128), lambda i, c: (0, c)),
            pl.BlockSpec((RB, NC * NS), lambda i, c: (i, 0)),
        ],
        out_specs=[
            pl.BlockSpec((1, RB, 128), lambda i, c: (c, i, 0)),
            pl.BlockSpec((RB, 128), lambda i, c: (i, 0)),
        ],
        out_shape=[
            jax.ShapeDtypeStruct((NC, N, 128), jnp.float32),
            jax.ShapeDtypeStruct((N, 128), jnp.float32),
        ],
    )(x, W1, degp)


def _k4_body(s1a_ref, s1b_ref, dinv_ref, b1_ref,
             w2a_ref, w2b_ref, y2_ref):
    dinv = dinv_ref[...]
    b1 = b1_ref[...]
    ha = jnp.maximum(dinv * s1a_ref[0] + b1[:, :128], 0.0)
    hb = jnp.maximum(dinv * s1b_ref[0] + b1[:, 128:], 0.0)
    u = (jnp.dot(ha.astype(jnp.bfloat16), w2a_ref[0],
                 preferred_element_type=jnp.float32)
         + jnp.dot(hb.astype(jnp.bfloat16), w2b_ref[0],
                   preferred_element_type=jnp.float32))
    y2_ref[...] = (dinv[:, :32] * u)[None]


def _k4(s1, dinvb, b1r, W2a, W2b):
    return pl.pallas_call(
        _k4_body,
        grid=(NRB, 2),
        in_specs=[
            pl.BlockSpec((1, RB, 128), lambda i, c: (0, i, 0)),
            pl.BlockSpec((1, RB, 128), lambda i, c: (1, i, 0)),
            pl.BlockSpec((RB, 128), lambda i, c: (i, 0)),
            pl.BlockSpec((1, D), lambda i, c: (0, 0)),
            pl.BlockSpec((1, 128, 32), lambda i, c: (c, 0, 0)),
            pl.BlockSpec((1, 128, 32), lambda i, c: (c, 0, 0)),
        ],
        out_specs=pl.BlockSpec((1, RB, 32), lambda i, c: (c, i, 0)),
        out_shape=jax.ShapeDtypeStruct((NC, N, 32), jnp.float32),
    )(s1, s1, dinvb, b1r, W2a, W2b)


def _k6_body(s2a_ref, s2b_ref, dinv_ref, b2_ref,
             wla_ref, wlb_ref, out_ref):
    dinv = dinv_ref[:, :32]
    b2 = b2_ref[...]
    h2a = dinv * s2a_ref[0] + b2[:, :32]
    h2b = dinv * s2b_ref[0] + b2[:, 32:]
    logits = (jnp.dot(h2a, wla_ref[...], preferred_element_type=jnp.float32)
              + jnp.dot(h2b, wlb_ref[...], preferred_element_type=jnp.float32))
    m = jnp.max(logits, axis=1, keepdims=True)
    ex = jnp.exp(logits - m)
    lse = jnp.log(jnp.sum(ex, axis=1, keepdims=True))
    out_ref[...] = logits - m - lse


def _k6(s2, dinvb, b2r, WlA, WlB):
    return pl.pallas_call(
        _k6_body,
        grid=(NRB,),
        in_specs=[
            pl.BlockSpec((1, RB, 32), lambda i: (0, i, 0)),
            pl.BlockSpec((1, RB, 32), lambda i: (1, i, 0)),
            pl.BlockSpec((RB, 128), lambda i: (i, 0)),
            pl.BlockSpec((1, CP), lambda i: (0, 0)),
            pl.BlockSpec((32, C), lambda i: (0, 0)),
            pl.BlockSpec((32, C), lambda i: (0, 0)),
        ],
        out_specs=pl.BlockSpec((RB, C), lambda i: (i, 0)),
        out_shape=jax.ShapeDtypeStruct((N, C), jnp.float32),
    )(s2, s2, dinvb, b2r, WlA, WlB)


def kernel(x, edge_index, W1, b1, W2, b2, Wlast):
    src = edge_index[0].astype(jnp.int32)
    dst = edge_index[1].astype(jnp.int32)

    zdeg = jnp.zeros((DEGP,), jnp.float32)

    b1r = b1.reshape(1, D)
    W2p = jnp.pad(W2, ((0, 0), (0, CP - C))).astype(jnp.bfloat16)
    W2a = W2p[:128].reshape(128, 2, 32).transpose(1, 0, 2)
    W2b = W2p[128:].reshape(128, 2, 32).transpose(1, 0, 2)
    b2r = jnp.pad(b2, (0, CP - C)).reshape(1, CP)
    Wlp = jnp.pad(Wlast, ((0, CP - C), (0, 0)))
    WlA, WlB = Wlp[:32], Wlp[32:]

    degp = _deg_kernel(dst, zdeg)
    degt = degp.T[:N]
    y1, dinvb = _k2(x.astype(jnp.bfloat16), W1.astype(jnp.bfloat16),
                    degt)
    s1 = _scatter128(y1, src, dst)
    y2 = _k4(s1, dinvb, b1r, W2a, W2b)
    s2 = _scatter32(y2, src, dst)
    return _k6(s2, dinvb, b2r, WlA, WlB)


# speedup vs baseline: 1.2559x; 1.2559x over previous
import functools

import jax
import jax.numpy as jnp
from jax import lax
from jax.experimental import pallas as pl
from jax.experimental.pallas import tpu as pltpu
from jax.experimental.pallas import tpu_sc as plsc

N = 10000
E = 160000
D = 256
H = 256
C = 40
CP = 64

NC = 2
NS = 16
DEGP = 10240
EB3, NB3 = 40, 5
EB5, NB5 = 80, 5

RB = 1000
NRB = N // RB

_mesh = plsc.VectorSubcoreMesh(core_axis_name="c", subcore_axis_name="s")

_sc_params = pltpu.CompilerParams(
    needs_layout_passes=False, use_tc_tiling_on_sc=False)


def _deg_body(dst_hbm, zdeg_hbm, out_hbm, idx_v, deg_v):
    c = lax.axis_index("c")
    s = lax.axis_index("s")
    wid = c * NS + s
    n_edges = E // (NC * NS)
    pltpu.sync_copy(zdeg_hbm, deg_v)
    idx_v[pl.ds(n_edges - 8, 16)] = jnp.zeros((16,), jnp.int32)
    pltpu.sync_copy(dst_hbm.at[pl.ds(wid * n_edges, n_edges)],
                    idx_v.at[pl.ds(0, n_edges)])
    ones = jnp.ones((16,), jnp.float32)
    iota = lax.iota(jnp.int32, 16)
    nblk = (n_edges + 15) // 16

    @pl.loop(0, nblk)
    def _(k):
        idx = idx_v[pl.ds(k * 16, 16)]
        mask = (k * 16 + iota) < n_edges
        plsc.addupdate_scatter(deg_v, [idx], ones, mask=mask)

    pltpu.sync_copy(deg_v, out_hbm.at[wid])


def _deg_kernel(dst32, zdeg):
    k = pl.kernel(
        _deg_body,
        out_type=jax.ShapeDtypeStruct((NC * NS, DEGP), jnp.float32),
        mesh=_mesh,
        compiler_params=_sc_params,
        scratch_types=[
            pltpu.VMEM((E // (NC * NS) + 8, ), jnp.int32),
            pltpu.VMEM((DEGP,), jnp.float32),
        ],
    )
    return k(dst32, zdeg)


def _make_scatter(F, eb, nb):
    rows_per_tile = N // NS
    ept = E // NS
    nblk = ept // eb
    assert nblk * eb == ept and nblk % nb == 0 and eb % 8 == 0

    def body(y_hbm, src_hbm, dst_hbm, out_hbm,
             src_v, dst_v, rows_v, acc_sh, *sems):
        c = lax.axis_index("c")
        s = lax.axis_index("s")
        yc = y_hbm.at[c]
        pltpu.sync_copy(yc.at[pl.ds(s * rows_per_tile, rows_per_tile)],
                        acc_sh.at[pl.ds(s * rows_per_tile, rows_per_tile)])
        pltpu.sync_copy(src_hbm.at[pl.ds(s * ept, ept)], src_v)
        pltpu.sync_copy(dst_hbm.at[pl.ds(s * ept, ept)], dst_v)
        plsc.subcore_barrier()

        for b in range(nb):
            pltpu.async_copy(
                yc.at[src_v.at[pl.ds(b * eb, eb)]], rows_v.at[b], sems[b])

        @pl.loop(0, nblk, step=nb)
        def _(j0):
            for b in range(nb):
                j = j0 + b
                pltpu.make_async_copy(
                    yc.at[src_v.at[pl.ds(j * eb, eb)]], rows_v.at[b],
                    sems[b]).wait()
                pltpu.sync_copy(rows_v.at[b],
                                acc_sh.at[dst_v.at[pl.ds(j * eb, eb)]],
                                add=True)

                @pl.when(j + nb < nblk)
                def _():
                    pltpu.async_copy(
                        yc.at[src_v.at[pl.ds((j + nb) * eb, eb)]],
                        rows_v.at[b], sems[b])

        plsc.subcore_barrier()
        pltpu.sync_copy(
            acc_sh.at[pl.ds(s * rows_per_tile, rows_per_tile)],
            out_hbm.at[c].at[pl.ds(s * rows_per_tile, rows_per_tile)])

    def run(y3, src, dst):
        k = pl.kernel(
            body,
            out_type=jax.ShapeDtypeStruct((NC, N, F), jnp.float32),
            mesh=_mesh,
            compiler_params=_sc_params,
            scratch_types=[
                pltpu.VMEM((ept,), jnp.int32),
                pltpu.VMEM((ept,), jnp.int32),
                pltpu.VMEM((nb, eb, F), jnp.float32),
                pltpu.VMEM_SHARED((N, F), jnp.float32),
            ] + [pltpu.SemaphoreType.DMA] * nb,
        )
        return k(y3, src, dst)

    return run


_scatter128 = _make_scatter(128, EB3, NB3)
_scatter32 = _make_scatter(32, EB5, NB5)


def _k2_body(x_ref, w1_ref, degp_ref, y_ref):
    deg = jnp.sum(degp_ref[...], axis=1, keepdims=True) + 1.0
    dinv = lax.rsqrt(deg)
    xw = jnp.dot(x_ref[...], w1_ref[...], preferred_element_type=jnp.float32)
    y_ref[...] = (xw * dinv)[None]


def _k2(x, W1, degp):
    return pl.pallas_call(
        _k2_body,
        grid=(NRB, 2),
        in_specs=[
            pl.BlockSpec((RB, D), lambda i, c: (i, 0)),
            pl.BlockSpec((D, 128), lambda i, c: (0, c)),
            pl.BlockSpec((RB, NC * NS), lambda i, c: (i, 0)),
        ],
        out_specs=pl.BlockSpec((1, RB, 128), lambda i, c: (c, i, 0)),
        out_shape=jax.ShapeDtypeStruct((NC, N, 128), jnp.float32),
    )(x, W1, degp)


def _k4_body(s1a_ref, s1b_ref, degp_ref, b1_ref,
             w2a_ref, w2b_ref, y2_ref):
    dinv = lax.rsqrt(jnp.sum(degp_ref[...], axis=1, keepdims=True) + 1.0)
    b1 = b1_ref[...]
    ha = jnp.maximum(dinv * s1a_ref[0] + b1[:, :128], 0.0)
    hb = jnp.maximum(dinv * s1b_ref[0] + b1[:, 128:], 0.0)
    u = (jnp.dot(ha.astype(jnp.bfloat16), w2a_ref[0],
                 preferred_element_type=jnp.float32)
         + jnp.dot(hb.astype(jnp.bfloat16), w2b_ref[0],
                   preferred_element_type=jnp.float32))
    y2_ref[...] = (dinv * u)[None]


def _k4(s1, degt, b1r, W2a, W2b):
    return pl.pallas_call(
        _k4_body,
        grid=(NRB, 2),
        in_specs=[
            pl.BlockSpec((1, RB, 128), lambda i, c: (0, i, 0)),
            pl.BlockSpec((1, RB, 128), lambda i, c: (1, i, 0)),
            pl.BlockSpec((RB, NC * NS), lambda i, c: (i, 0)),
            pl.BlockSpec((1, D), lambda i, c: (0, 0)),
            pl.BlockSpec((1, 128, 32), lambda i, c: (c, 0, 0)),
            pl.BlockSpec((1, 128, 32), lambda i, c: (c, 0, 0)),
        ],
        out_specs=pl.BlockSpec((1, RB, 32), lambda i, c: (c, i, 0)),
        out_shape=jax.ShapeDtypeStruct((NC, N, 32), jnp.float32),
    )(s1, s1, degt, b1r, W2a, W2b)


def _k6_body(s2a_ref, s2b_ref, degp_ref, b2_ref,
             wla_ref, wlb_ref, out_ref):
    dinv = lax.rsqrt(jnp.sum(degp_ref[...], axis=1, keepdims=True) + 1.0)
    b2 = b2_ref[...]
    h2a = dinv * s2a_ref[0] + b2[:, :32]
    h2b = dinv * s2b_ref[0] + b2[:, 32:]
    logits = (jnp.dot(h2a, wla_ref[...], preferred_element_type=jnp.float32)
              + jnp.dot(h2b, wlb_ref[...], preferred_element_type=jnp.float32))
    m = jnp.max(logits, axis=1, keepdims=True)
    ex = jnp.exp(logits - m)
    lse = jnp.log(jnp.sum(ex, axis=1, keepdims=True))
    out_ref[...] = logits - m - lse


def _k6(s2, degt, b2r, WlA, WlB):
    return pl.pallas_call(
        _k6_body,
        grid=(NRB,),
        in_specs=[
            pl.BlockSpec((1, RB, 32), lambda i: (0, i, 0)),
            pl.BlockSpec((1, RB, 32), lambda i: (1, i, 0)),
            pl.BlockSpec((RB, NC * NS), lambda i: (i, 0)),
            pl.BlockSpec((1, CP), lambda i: (0, 0)),
            pl.BlockSpec((32, C), lambda i: (0, 0)),
            pl.BlockSpec((32, C), lambda i: (0, 0)),
        ],
        out_specs=pl.BlockSpec((RB, C), lambda i: (i, 0)),
        out_shape=jax.ShapeDtypeStruct((N, C), jnp.float32),
    )(s2, s2, degt, b2r, WlA, WlB)


def kernel(x, edge_index, W1, b1, W2, b2, Wlast):
    src = edge_index[0].astype(jnp.int32)
    dst = edge_index[1].astype(jnp.int32)

    zdeg = jnp.zeros((DEGP,), jnp.float32)

    b1r = b1.reshape(1, D)
    W2p = jnp.pad(W2, ((0, 0), (0, CP - C))).astype(jnp.bfloat16)
    W2a = W2p[:128].reshape(128, 2, 32).transpose(1, 0, 2)
    W2b = W2p[128:].reshape(128, 2, 32).transpose(1, 0, 2)
    b2r = jnp.pad(b2, (0, CP - C)).reshape(1, CP)
    Wlp = jnp.pad(Wlast, ((0, CP - C), (0, 0)))
    WlA, WlB = Wlp[:32], Wlp[32:]

    degp = _deg_kernel(dst, zdeg)
    degt = degp.T[:N]
    y1 = _k2(x.astype(jnp.bfloat16), W1.astype(jnp.bfloat16),
             degt)
    s1 = _scatter128(y1, src, dst)
    y2 = _k4(s1, degt, b1r, W2a, W2b)
    s2 = _scatter32(y2, src, dst)
    return _k6(s2, degt, b2r, WlA, WlB)
